# baseline (speedup 1.0000x reference)
import functools

import jax
import jax.numpy as jnp
from jax import lax
from jax.experimental import pallas as pl
from jax.experimental.pallas import tpu as pltpu
from jax.experimental.pallas import tpu_sc as plsc

_ROW_BLOCK = 1024


def _argmin_body(xsq_ref, x_ref, keysm2_ref, ksq_ref, values_ref, idx_ref, y_ref):
    x = x_ref[...]
    keysm2 = keysm2_ref[...]
    dot = lax.dot_general(
        x, keysm2, (((1,), (1,)), ((), ())),
        preferred_element_type=jnp.float32)
    xsq = xsq_ref[...].reshape(_ROW_BLOCK, 1)
    d2 = xsq + dot + ksq_ref[...]
    dist = jnp.sqrt(jnp.maximum(d2, 0.0))
    idx = jnp.argmin(dist, axis=1).astype(jnp.int32)
    idx_ref[...] = idx
    k = keysm2.shape[0]
    onehot = (idx[:, None] == lax.broadcasted_iota(jnp.int32, (_ROW_BLOCK, k), 1)
              ).astype(jnp.float32)
    y_ref[...] = lax.dot_general(
        onehot, values_ref[...], (((1,), (0,)), ((), ())),
        precision=lax.Precision.HIGHEST,
        preferred_element_type=jnp.float32)


def _nearest_code(flat, xsq, keys, ksq, values):
    n = flat.shape[0]
    k = keys.shape[0]
    out_c = values.shape[1]
    grid = n // _ROW_BLOCK
    return pl.pallas_call(
        _argmin_body,
        grid=(grid,),
        in_specs=[
            pl.BlockSpec((_ROW_BLOCK,), lambda i: (i,)),
            pl.BlockSpec((_ROW_BLOCK, flat.shape[1]), lambda i: (i, 0)),
            pl.BlockSpec((k, keys.shape[1]), lambda i: (0, 0)),
            pl.BlockSpec((1, k), lambda i: (0, 0)),
            pl.BlockSpec((k, out_c), lambda i: (0, 0)),
        ],
        out_specs=[
            pl.BlockSpec((_ROW_BLOCK,), lambda i: (i,)),
            pl.BlockSpec((_ROW_BLOCK, out_c), lambda i: (i, 0)),
        ],
        out_shape=[
            jax.ShapeDtypeStruct((n,), jnp.int32),
            jax.ShapeDtypeStruct((n, out_c), jnp.float32),
        ],
    )(xsq, flat, keys, ksq, values)


def _gather_rows(values, idx):
    n = idx.shape[0]
    out_c = values.shape[1]
    mesh = plsc.VectorSubcoreMesh(core_axis_name="c", subcore_axis_name="s")
    num_workers = 2 * 16
    b_per_w = n // num_workers
    sub = 96
    n_sub = b_per_w // sub

    @functools.partial(
        pl.kernel,
        mesh=mesh,
        out_type=jax.ShapeDtypeStruct((n, out_c), jnp.float32),
        scratch_types=[
            pltpu.VMEM((b_per_w,), jnp.int32),
            pltpu.VMEM((2, sub, out_c), jnp.float32),
            pltpu.SemaphoreType.DMA((2,)),
            pltpu.SemaphoreType.DMA((2,)),
        ],
    )
    def gather_kernel(values_hbm, idx_hbm, out_hbm, idx_v, rows_v, gsem, ssem):
        wid = lax.axis_index("s") * 2 + lax.axis_index("c")
        base = wid * b_per_w
        pltpu.sync_copy(idx_hbm.at[pl.ds(base, b_per_w)], idx_v)
        gathers = [None] * n_sub
        stores = [None] * n_sub
        gathers[0] = pltpu.async_copy(
            values_hbm.at[idx_v.at[pl.ds(0, sub)]], rows_v.at[0], gsem.at[0])
        for j in range(n_sub):
            nxt = j + 1
            if nxt < n_sub:
                if j >= 1:
                    stores[j - 1].wait()
                gathers[nxt] = pltpu.async_copy(
                    values_hbm.at[idx_v.at[pl.ds(nxt * sub, sub)]],
                    rows_v.at[nxt % 2], gsem.at[nxt % 2])
            gathers[j].wait()
            stores[j] = pltpu.async_copy(
                rows_v.at[j % 2],
                out_hbm.at[pl.ds(base + j * sub, sub)], ssem.at[j % 2])
        stores[n_sub - 2].wait()
        stores[n_sub - 1].wait()

    return gather_kernel(values, idx)


@jax.jit
def kernel(x, keys, values):
    batchsz, lenseq, in_c = x.shape
    n = batchsz * lenseq
    flat = lax.optimization_barrier(x.reshape(n, in_c))
    xsq = jnp.sum(flat * flat, axis=1)
    ksq = jnp.sum(keys * keys, axis=1)[None, :]
    idx, y = _nearest_code(flat, xsq, keys * (-2.0), ksq, values)
    return y.reshape(batchsz, lenseq, values.shape[-1])


# speedup vs baseline: 1.2507x; 1.2507x over previous
import functools

import jax
import jax.numpy as jnp
from jax import lax
from jax.experimental import pallas as pl
from jax.experimental.pallas import tpu as pltpu
from jax.experimental.pallas import tpu_sc as plsc

_ROW_BLOCK = 1024


def _argmin_body(xsq_ref, x_ref, keysm2_ref, ksq_ref, vhi_ref, vmid_ref,
                 vlo_ref, idx_ref, y_ref):
    x = x_ref[...]
    keysm2 = keysm2_ref[...]
    dot = lax.dot_general(
        x, keysm2, (((1,), (1,)), ((), ())),
        preferred_element_type=jnp.float32)
    xsq = xsq_ref[...].reshape(_ROW_BLOCK, 1)
    d2 = xsq + dot + ksq_ref[...]
    dist = jnp.sqrt(jnp.maximum(d2, 0.0))
    idx = jnp.argmin(dist, axis=1).astype(jnp.int32)
    idx_ref[...] = idx
    k = keysm2.shape[0]
    onehot = (idx[:, None] == lax.broadcasted_iota(jnp.int32, (_ROW_BLOCK, k), 1)
              ).astype(jnp.bfloat16)
    parts = [
        lax.dot_general(onehot, v_ref[...], (((1,), (0,)), ((), ())),
                        preferred_element_type=jnp.float32)
        for v_ref in (vhi_ref, vmid_ref, vlo_ref)
    ]
    y_ref[...] = (parts[0] + parts[1]) + parts[2]


def _nearest_code(flat, xsq, keys, ksq, vhi, vmid, vlo):
    n = flat.shape[0]
    k = keys.shape[0]
    out_c = vhi.shape[1]
    grid = n // _ROW_BLOCK
    vspec = pl.BlockSpec((k, out_c), lambda i: (0, 0))
    return pl.pallas_call(
        _argmin_body,
        grid=(grid,),
        in_specs=[
            pl.BlockSpec((_ROW_BLOCK,), lambda i: (i,)),
            pl.BlockSpec((_ROW_BLOCK, flat.shape[1]), lambda i: (i, 0)),
            pl.BlockSpec((k, keys.shape[1]), lambda i: (0, 0)),
            pl.BlockSpec((1, k), lambda i: (0, 0)),
            vspec, vspec, vspec,
        ],
        out_specs=[
            pl.BlockSpec((_ROW_BLOCK,), lambda i: (i,)),
            pl.BlockSpec((_ROW_BLOCK, out_c), lambda i: (i, 0)),
        ],
        out_shape=[
            jax.ShapeDtypeStruct((n,), jnp.int32),
            jax.ShapeDtypeStruct((n, out_c), jnp.float32),
        ],
    )(xsq, flat, keys, ksq, vhi, vmid, vlo)


def _gather_rows(values, idx):
    n = idx.shape[0]
    out_c = values.shape[1]
    mesh = plsc.VectorSubcoreMesh(core_axis_name="c", subcore_axis_name="s")
    num_workers = 2 * 16
    b_per_w = n // num_workers
    sub = 96
    n_sub = b_per_w // sub

    @functools.partial(
        pl.kernel,
        mesh=mesh,
        out_type=jax.ShapeDtypeStruct((n, out_c), jnp.float32),
        scratch_types=[
            pltpu.VMEM((b_per_w,), jnp.int32),
            pltpu.VMEM((2, sub, out_c), jnp.float32),
            pltpu.SemaphoreType.DMA((2,)),
            pltpu.SemaphoreType.DMA((2,)),
        ],
    )
    def gather_kernel(values_hbm, idx_hbm, out_hbm, idx_v, rows_v, gsem, ssem):
        wid = lax.axis_index("s") * 2 + lax.axis_index("c")
        base = wid * b_per_w
        pltpu.sync_copy(idx_hbm.at[pl.ds(base, b_per_w)], idx_v)
        gathers = [None] * n_sub
        stores = [None] * n_sub
        gathers[0] = pltpu.async_copy(
            values_hbm.at[idx_v.at[pl.ds(0, sub)]], rows_v.at[0], gsem.at[0])
        for j in range(n_sub):
            nxt = j + 1
            if nxt < n_sub:
                if j >= 1:
                    stores[j - 1].wait()
                gathers[nxt] = pltpu.async_copy(
                    values_hbm.at[idx_v.at[pl.ds(nxt * sub, sub)]],
                    rows_v.at[nxt % 2], gsem.at[nxt % 2])
            gathers[j].wait()
            stores[j] = pltpu.async_copy(
                rows_v.at[j % 2],
                out_hbm.at[pl.ds(base + j * sub, sub)], ssem.at[j % 2])
        stores[n_sub - 2].wait()
        stores[n_sub - 1].wait()

    return gather_kernel(values, idx)


@jax.jit
def kernel(x, keys, values):
    batchsz, lenseq, in_c = x.shape
    n = batchsz * lenseq
    flat = lax.optimization_barrier(x.reshape(n, in_c))
    xsq = jnp.sum(flat * flat, axis=1)
    ksq = jnp.sum(keys * keys, axis=1)[None, :]
    vhi = values.astype(jnp.bfloat16)
    r1 = values - vhi.astype(jnp.float32)
    vmid = r1.astype(jnp.bfloat16)
    vlo = (r1 - vmid.astype(jnp.float32)).astype(jnp.bfloat16)
    idx, y = _nearest_code(flat, xsq, keys * (-2.0), ksq, vhi, vmid, vlo)
    return y.reshape(batchsz, lenseq, values.shape[-1])
